# Initial kernel scaffold; baseline (speedup 1.0000x reference)
import jax
import jax.numpy as jnp
from jax.experimental import pallas as pl


def kernel(x, edge_index, W, b):
    raise NotImplementedError("write your pallas kernel here")


# speedup vs baseline: 12.3848x; 12.3848x over previous
import functools

import jax
import jax.numpy as jnp
from jax import lax
from jax.experimental import pallas as pl
from jax.experimental.pallas import tpu as pltpu
from jax.experimental.pallas import tpu_sc as plsc

N = 10000
E = 320000
D = 128

NC = 2
NS = 16
NW = NC * NS
CHUNK = 128
NCH = 80
EPW = NCH * CHUNK
EPAD = NW * EPW
NPAD = 10240
RPT = NPAD // NS
SENT = N

_mesh = plsc.VectorSubcoreMesh(core_axis_name="c", subcore_axis_name="s")


@functools.partial(
    pl.kernel,
    mesh=_mesh,
    out_type=jax.ShapeDtypeStruct((NC, NPAD), jnp.float32),
    scratch_types=[
        pltpu.VMEM((2, CHUNK), jnp.int32),
        pltpu.VMEM((CHUNK,), jnp.float32),
        pltpu.VMEM_SHARED((NPAD,), jnp.float32),
        pltpu.SemaphoreType.DMA,
    ],
)
def _deg_sc(ed_hbm, ones_hbm, zeros_hbm, degp_hbm, ed_v, ones_v, deg_sh, sem):
    cid = lax.axis_index("c")
    sid = lax.axis_index("s")
    wid = cid * NS + sid
    pltpu.sync_copy(ones_hbm, ones_v)
    pltpu.sync_copy(zeros_hbm.at[pl.ds(sid * RPT, RPT)],
                    deg_sh.at[pl.ds(sid * RPT, RPT)])
    plsc.subcore_barrier()

    def body(j, carry):
        pltpu.sync_copy(ed_hbm.at[wid * NCH + j], ed_v)
        pltpu.sync_copy(ones_v, deg_sh.at[ed_v.at[1]], add=True)
        return carry

    lax.fori_loop(0, NCH, body, 0)
    plsc.subcore_barrier()
    pltpu.sync_copy(deg_sh.at[pl.ds(sid * RPT, RPT)],
                    degp_hbm.at[cid, pl.ds(sid * RPT, RPT)])


def _lin_body(x_ref, w_ref, degp_ref, g_ref, dinv_ref):
    h = jnp.dot(x_ref[...], w_ref[...], preferred_element_type=jnp.float32)
    deg = degp_ref[0] + degp_ref[1] + 1.0
    dinv = lax.rsqrt(deg)
    dinv_ref[...] = dinv
    g_ref[...] = h * dinv


_ROWB = 1280


def _linear(x_p, W, degp):
    grid = NPAD // _ROWB
    return pl.pallas_call(
        _lin_body,
        grid=(grid,),
        in_specs=[
            pl.BlockSpec((_ROWB, D), lambda i: (i, 0)),
            pl.BlockSpec((D, D), lambda i: (0, 0)),
            pl.BlockSpec((NC, _ROWB, 1), lambda i: (0, i, 0)),
        ],
        out_specs=[
            pl.BlockSpec((_ROWB, D), lambda i: (i, 0)),
            pl.BlockSpec((_ROWB, 1), lambda i: (i, 0)),
        ],
        out_shape=[
            jax.ShapeDtypeStruct((NPAD, D), jnp.float32),
            jax.ShapeDtypeStruct((NPAD, 1), jnp.float32),
        ],
    )(x_p, W, degp)


@functools.partial(
    pl.kernel,
    mesh=_mesh,
    out_type=jax.ShapeDtypeStruct((NC, NPAD, D), jnp.float32),
    scratch_types=[
        pltpu.VMEM((2, CHUNK), jnp.int32),
        pltpu.VMEM((2, CHUNK), jnp.int32),
        pltpu.VMEM((CHUNK, D), jnp.float32),
        pltpu.VMEM((CHUNK, D), jnp.float32),
        pltpu.VMEM_SHARED((NPAD, D), jnp.float32),
        pltpu.SemaphoreType.DMA,
        pltpu.SemaphoreType.DMA,
    ],
)
def _agg_sc(ed_hbm, g_hbm, zeros_hbm, aggp_hbm,
            ed_a, ed_b, rows_a, rows_b, agg_sh, sem_a, sem_b):
    cid = lax.axis_index("c")
    sid = lax.axis_index("s")
    wid = cid * NS + sid
    base = wid * NCH
    pltpu.sync_copy(zeros_hbm.at[pl.ds(sid * RPT, RPT)],
                    agg_sh.at[pl.ds(sid * RPT, RPT)])
    plsc.subcore_barrier()

    pltpu.sync_copy(ed_hbm.at[base], ed_a)
    pltpu.async_copy(g_hbm.at[ed_a.at[0]], rows_a, sem_a)

    def body(i, carry):
        j = i * 2
        pltpu.sync_copy(ed_hbm.at[base + j + 1], ed_b)
        pltpu.async_copy(g_hbm.at[ed_b.at[0]], rows_b, sem_b)
        pltpu.make_async_copy(g_hbm.at[ed_a.at[0]], rows_a, sem_a).wait()
        pltpu.sync_copy(rows_a, agg_sh.at[ed_a.at[1]], add=True)
        pltpu.sync_copy(ed_hbm.at[base + (j + 2) % NCH], ed_a)
        pltpu.async_copy(g_hbm.at[ed_a.at[0]], rows_a, sem_a)
        pltpu.make_async_copy(g_hbm.at[ed_b.at[0]], rows_b, sem_b).wait()
        pltpu.sync_copy(rows_b, agg_sh.at[ed_b.at[1]], add=True)
        return carry

    lax.fori_loop(0, NCH // 2, body, 0)
    pltpu.make_async_copy(g_hbm.at[ed_a.at[0]], rows_a, sem_a).wait()
    plsc.subcore_barrier()
    pltpu.sync_copy(agg_sh.at[pl.ds(sid * RPT, RPT)],
                    aggp_hbm.at[cid, pl.ds(sid * RPT, RPT)])


def _out_body(aggp_ref, g_ref, dinv_ref, b_ref, o_ref):
    s = aggp_ref[0] + aggp_ref[1] + g_ref[...]
    o_ref[...] = jnp.maximum(s * dinv_ref[...] + b_ref[...], 0.0)


def _epilogue(aggp, g, dinv, b2):
    grid = NPAD // _ROWB
    return pl.pallas_call(
        _out_body,
        grid=(grid,),
        in_specs=[
            pl.BlockSpec((NC, _ROWB, D), lambda i: (0, i, 0)),
            pl.BlockSpec((_ROWB, D), lambda i: (i, 0)),
            pl.BlockSpec((_ROWB, 1), lambda i: (i, 0)),
            pl.BlockSpec((1, D), lambda i: (0, 0)),
        ],
        out_specs=pl.BlockSpec((_ROWB, D), lambda i: (i, 0)),
        out_shape=jax.ShapeDtypeStruct((NPAD, D), jnp.float32),
    )(aggp, g, dinv, b2)


def kernel(x, edge_index, W, b):
    pad = EPAD - E
    src_p = jnp.concatenate(
        [edge_index[0], jnp.full((pad,), SENT, jnp.int32)]).reshape(NW * NCH, 1, CHUNK)
    dst_p = jnp.concatenate(
        [edge_index[1], jnp.full((pad,), SENT, jnp.int32)]).reshape(NW * NCH, 1, CHUNK)
    ed_p = jnp.concatenate([src_p, dst_p], axis=1)
    x_p = jnp.zeros((NPAD, D), jnp.float32).at[:N].set(x)

    ones = jnp.ones((CHUNK,), jnp.float32)
    zeros_n = jnp.zeros((NPAD,), jnp.float32)
    zeros_nd = jnp.zeros((NPAD, D), jnp.float32)

    degp = _deg_sc(ed_p, ones, zeros_n)
    g, dinv = _linear(x_p, W, degp.reshape(NC, NPAD, 1))
    aggp = _agg_sc(ed_p, g, zeros_nd)
    out = _epilogue(aggp, g, dinv, b.reshape(1, D))
    return out[:N]
